# baseline (speedup 1.0000x reference)
import functools

import jax
import jax.numpy as jnp
from jax import lax
from jax.experimental import pallas as pl
from jax.experimental.pallas import tpu as pltpu
from jax.experimental.pallas import tpu_sc as plsc

N, T, D = 1024, 65, 768
SEGS = ((0, 0, 1), (18, 1, 4), (26, 5, 4), (34, 9, 4), (42, 13, 4))
T_OUT = 17
NUM_WORKERS = 32
N_PER_W = N // NUM_WORKERS

_mesh = plsc.VectorSubcoreMesh(core_axis_name="c", subcore_axis_name="s")


@functools.partial(
    pl.kernel,
    mesh=_mesh,
    out_type=jax.ShapeDtypeStruct((N, T_OUT * D), jnp.float32),
    scratch_types=[pltpu.SemaphoreType.DMA],
)
def _patch_drop(x_hbm, out_hbm, sem):
    wid = lax.axis_index("s") * 2 + lax.axis_index("c")
    n0 = wid * N_PER_W
    copies = []
    for src, dst, ln in SEGS:
        copies.append(
            pltpu.async_copy(
                x_hbm.at[pl.ds(n0, N_PER_W), pl.ds(src * D, ln * D)],
                out_hbm.at[pl.ds(n0, N_PER_W), pl.ds(dst * D, ln * D)],
                sem,
            )
        )
    for c in copies:
        c.wait()


def kernel(x):
    out = _patch_drop(x.reshape(N, T * D))
    return out.reshape(N, T_OUT, D)


# speedup vs baseline: 4.1713x; 4.1713x over previous
import functools

import jax
import jax.numpy as jnp
from jax import lax
from jax.experimental import pallas as pl
from jax.experimental.pallas import tpu as pltpu
from jax.experimental.pallas import tpu_sc as plsc

N, T, D = 1024, 65, 768
SEGS = ((0, 0, 1), (18, 1, 4), (26, 5, 4), (34, 9, 4), (42, 13, 4))
T_OUT = 17
NUM_WORKERS = 32
N_PER_W = N // NUM_WORKERS

_mesh = plsc.VectorSubcoreMesh(core_axis_name="c", subcore_axis_name="s")

CHUNK = 4
NCHUNKS = N_PER_W // CHUNK


@functools.partial(
    pl.kernel,
    mesh=_mesh,
    out_type=jax.ShapeDtypeStruct((N, T_OUT * D), jnp.float32),
    scratch_types=[
        pltpu.VMEM((2, CHUNK, T_OUT * D), jnp.float32),
        pltpu.SemaphoreType.DMA,
        pltpu.SemaphoreType.DMA,
        pltpu.SemaphoreType.DMA,
        pltpu.SemaphoreType.DMA,
    ],
)
def _patch_drop(x_hbm, out_hbm, buf, gsem0, gsem1, wsem0, wsem1):
    wid = lax.axis_index("s") * 2 + lax.axis_index("c")
    n0 = wid * N_PER_W
    gsems = (gsem0, gsem1)
    wsems = (wsem0, wsem1)
    gh = [None, None]
    wh = [None, None]
    for c in range(NCHUNKS + 1):
        slot = c % 2
        if c < NCHUNKS:
            if wh[slot] is not None:
                wh[slot].wait()
            n = n0 + c * CHUNK
            gh[slot] = [
                pltpu.async_copy(
                    x_hbm.at[pl.ds(n, CHUNK), pl.ds(src * D, ln * D)],
                    buf.at[slot, :, pl.ds(dst * D, ln * D)],
                    gsems[slot],
                )
                for src, dst, ln in SEGS
            ]
        if c >= 1:
            pslot = (c - 1) % 2
            for h in gh[pslot]:
                h.wait()
            np_ = n0 + (c - 1) * CHUNK
            wh[pslot] = pltpu.async_copy(
                buf.at[pslot], out_hbm.at[pl.ds(np_, CHUNK)], wsems[pslot]
            )
    for slot in (0, 1):
        if wh[slot] is not None:
            wh[slot].wait()


def kernel(x):
    out = _patch_drop(x.reshape(N, T * D))
    return out.reshape(N, T_OUT, D)


# speedup vs baseline: 7.2273x; 1.7326x over previous
import functools

import jax
import jax.numpy as jnp
import numpy as np
from jax import lax
from jax.experimental import pallas as pl
from jax.experimental.pallas import tpu as pltpu
from jax.experimental.pallas import tpu_sc as plsc

N, T, D = 1024, 65, 768
T_OUT = 17
NUM_WORKERS = 32
N_PER_W = N // NUM_WORKERS

NSLOTS = 6
LAG = 3

_KEEP = np.array(
    [0] + [1 + r * 8 + c for r in range(2, 6) for c in range(1, 5)],
    dtype=np.int32,
)

_mesh = plsc.VectorSubcoreMesh(core_axis_name="c", subcore_axis_name="s")


@functools.partial(
    pl.kernel,
    mesh=_mesh,
    out_type=jax.ShapeDtypeStruct((N, T_OUT, D), jnp.float32),
    scratch_types=[
        pltpu.VMEM((T_OUT,), jnp.int32),
        pltpu.VMEM((NSLOTS, T_OUT, D), jnp.float32),
        [pltpu.SemaphoreType.DMA] * NSLOTS,
        [pltpu.SemaphoreType.DMA] * NSLOTS,
    ],
)
def _patch_drop(x_hbm, idx_hbm, out_hbm, idx_v, buf, gsems, wsems):
    wid = lax.axis_index("s") * 2 + lax.axis_index("c")
    n0 = wid * N_PER_W
    pltpu.sync_copy(idx_hbm, idx_v)
    gh = [None] * NSLOTS
    wh = [None] * NSLOTS
    for i in range(N_PER_W + LAG):
        s = i % NSLOTS
        if i < N_PER_W:
            if wh[s] is not None:
                wh[s].wait()
            gh[s] = pltpu.async_copy(
                x_hbm.at[n0 + i].at[idx_v], buf.at[s], gsems[s]
            )
        j = i - LAG
        if j >= 0:
            sj = j % NSLOTS
            gh[sj].wait()
            wh[sj] = pltpu.async_copy(
                buf.at[sj], out_hbm.at[n0 + j], wsems[sj]
            )
    for s in range(NSLOTS):
        if wh[s] is not None:
            wh[s].wait()


def kernel(x):
    return _patch_drop(x, jnp.asarray(_KEEP))


# speedup vs baseline: 7.3508x; 1.0171x over previous
import functools

import jax
import jax.numpy as jnp
import numpy as np
from jax import lax
from jax.experimental import pallas as pl
from jax.experimental.pallas import tpu as pltpu
from jax.experimental.pallas import tpu_sc as plsc

N, T, D = 1024, 65, 768
T_OUT = 17
NUM_WORKERS = 32
N_PER_W = N // NUM_WORKERS

T_PAD = 18
_GATHER_IDX = np.array(
    [0] + [1 + r * 8 + c for r in range(2, 6) for c in range(1, 5)] + [45],
    dtype=np.int32,
)
_SCATTER_IDX = np.array(list(range(T_OUT)) + [T_OUT - 1], dtype=np.int32)

NSLOTS = 6
LAG = 3

_mesh = plsc.VectorSubcoreMesh(core_axis_name="c", subcore_axis_name="s")


@functools.partial(
    pl.kernel,
    mesh=_mesh,
    out_type=jax.ShapeDtypeStruct((N, T_OUT, D), jnp.float32),
    scratch_types=[
        pltpu.VMEM((T_PAD,), jnp.int32),
        pltpu.VMEM((T_PAD,), jnp.int32),
        pltpu.VMEM((NSLOTS, T_PAD, D), jnp.float32),
        [pltpu.SemaphoreType.DMA] * NSLOTS,
        [pltpu.SemaphoreType.DMA] * NSLOTS,
    ],
)
def _patch_drop(x_hbm, gidx_hbm, sidx_hbm, out_hbm, gidx_v, sidx_v, buf,
                gsems, wsems):
    wid = lax.axis_index("s") * 2 + lax.axis_index("c")
    n0 = wid * N_PER_W
    pltpu.sync_copy(gidx_hbm, gidx_v)
    pltpu.sync_copy(sidx_hbm, sidx_v)
    gh = [None] * NSLOTS
    wh = [None] * NSLOTS
    for i in range(N_PER_W + LAG):
        s = i % NSLOTS
        if i < N_PER_W:
            if wh[s] is not None:
                wh[s].wait()
            gh[s] = pltpu.async_copy(
                x_hbm.at[n0 + i].at[gidx_v], buf.at[s], gsems[s]
            )
        j = i - LAG
        if j >= 0:
            sj = j % NSLOTS
            gh[sj].wait()
            wh[sj] = pltpu.async_copy(
                buf.at[sj], out_hbm.at[n0 + j].at[sidx_v], wsems[sj]
            )
    for s in range(NSLOTS):
        if wh[s] is not None:
            wh[s].wait()


def kernel(x):
    return _patch_drop(x, jnp.asarray(_GATHER_IDX), jnp.asarray(_SCATTER_IDX))
